# baseline (speedup 1.0000x reference)
import functools

import jax
import jax.numpy as jnp
from jax import lax
from jax.experimental import pallas as pl
from jax.experimental.pallas import tpu as pltpu
from jax.experimental.pallas import tpu_sc as plsc

_K = 8
_NEG = -1e30
_NPATH = 8


def _topk_kernel(x_ref, idx_ref, xn_ref, *, blk, pad_idx):
    i = pl.program_id(1)
    b = pl.program_id(0)
    t_total = x_ref.shape[1]

    @pl.when(i == 0)
    def _normalize():
        xa_full = x_ref[0]
        n2 = jnp.sum(xa_full * xa_full, axis=1, keepdims=True)
        xn_ref[...] = xa_full / (jnp.sqrt(n2) + 1e-8)

    row0 = i * blk

    def _path(wcols):
        xn_rows = xn_ref[pl.ds(row0, blk), :]
        sim = jax.lax.dot_general(
            xn_rows, xn_ref[:wcols, :], (((1,), (1,)), ((), ())),
            preferred_element_type=jnp.float32)

        cols = jax.lax.broadcasted_iota(jnp.int32, (blk, wcols), 1)
        rows = row0 + jax.lax.broadcasted_iota(jnp.int32, (blk, wcols), 0)
        w = jnp.where(cols <= rows, sim, _NEG)

        deg = jnp.zeros((blk, 1), jnp.int32)
        m = None
        for r in range(_K):
            t = w if m is None else jnp.where(w < m, w, _NEG)
            m = jnp.max(t, axis=1, keepdims=True)
            am = jnp.min(jnp.where(t == m, cols, jnp.int32(1 << 30)),
                         axis=1, keepdims=True)
            valid = m > -1e29
            gidx = jnp.where(valid, b * t_total + am, jnp.int32(pad_idx))
            idx_ref[0, :, r:r + 1] = gidx
            deg = deg + valid.astype(jnp.int32)
        idx_ref[0, :, _K:_K + 1] = deg

    nblk = t_total // blk
    npath = _NPATH if nblk % _NPATH == 0 and nblk >= _NPATH else 1
    per_path = nblk // npath
    for p in range(npath):
        lo, hi = p * per_path, (p + 1) * per_path
        cond = (i >= lo) & (i < hi) if p else (i < hi)

        @pl.when(cond)
        def _run(p=p):
            _path((p + 1) * per_path * blk)


def _epilogue_kernel(x_ref, ms_ref, idx_ref, gain_ref, bias_ref, lm_ref,
                     ls_ref, out_ref):
    deg = jnp.maximum(idx_ref[0, :, _K:_K + 1].astype(jnp.float32), 1.0)
    msg = ms_ref[0] / deg
    mix = jax.nn.sigmoid(lm_ref[0, 0])
    scale = jax.nn.softplus(ls_ref[0, 0]) + 0.01
    blended = mix * x_ref[0] + (1.0 - mix) * msg
    y = blended * gain_ref[0][None, :] + bias_ref[0][None, :]
    gelu = 0.5 * y * (1.0 + jax.lax.erf(y * (2.0 ** -0.5)))
    out_ref[0] = gelu * scale


def kernel(x, gain, bias, log_mix, log_scale):
    B, T, D = x.shape
    BT = B * T
    blk = min(256, T)
    grid = (B, T // blk)

    idx = pl.pallas_call(
        functools.partial(_topk_kernel, blk=blk, pad_idx=BT),
        grid=grid,
        in_specs=[pl.BlockSpec((1, T, D), lambda bb, ii: (bb, 0, 0))],
        out_specs=pl.BlockSpec((1, blk, 16), lambda bb, ii: (bb, ii, 0)),
        out_shape=jax.ShapeDtypeStruct((B, T, 16), jnp.int32),
        scratch_shapes=[pltpu.VMEM((T, D), jnp.float32)],
    )(x)

    x2d = jnp.concatenate(
        [x.reshape(BT, D), jnp.zeros((8, D), x.dtype)], axis=0)
    idx_flat = idx[:, :, :_K].reshape(BT * _K)

    NC, NS = 2, 16
    NW = NC * NS
    ntok_w = BT // NW
    CH = 8
    nchunk = ntok_w // CH

    @functools.partial(
        pl.kernel,
        out_type=jax.ShapeDtypeStruct((BT, D), jnp.float32),
        mesh=plsc.VectorSubcoreMesh(core_axis_name="c", subcore_axis_name="s"),
        scratch_types=[
            pltpu.VMEM((ntok_w * _K,), jnp.int32),
            pltpu.VMEM((CH * _K, D), jnp.float32),
            pltpu.VMEM((CH, D), jnp.float32),
            pltpu.SemaphoreType.DMA,
        ],
    )
    def _sc_gather(idx_hbm, x2d_hbm, out_hbm, idx_v, rows_v, acc_v, sem):
        wid = lax.axis_index("s") * NC + lax.axis_index("c")
        base = wid * ntok_w
        pltpu.sync_copy(idx_hbm.at[pl.ds(base * _K, ntok_w * _K)], idx_v)

        def chunk_body(c, carry):
            pltpu.async_copy(
                x2d_hbm.at[idx_v.at[pl.ds(c * (CH * _K), CH * _K)]],
                rows_v, sem).wait()

            def tok_body(t, carry2):
                def d_body(d, carry3):
                    s = pl.ds(d * 16, 16)
                    acc = rows_v[t * _K + 0, s]
                    for r in range(1, _K):
                        acc = acc + rows_v[t * _K + r, s]
                    acc_v[t, s] = acc
                    return carry3
                return lax.fori_loop(0, D // 16, d_body, carry2)

            lax.fori_loop(0, CH, tok_body, 0)
            pltpu.sync_copy(acc_v, out_hbm.at[pl.ds(base + c * CH, CH)])
            return carry

        lax.fori_loop(0, nchunk, chunk_body, 0)

    msgsum = _sc_gather(idx_flat, x2d).reshape(B, T, D)

    eblk = min(512, T)
    return pl.pallas_call(
        _epilogue_kernel,
        grid=(B, T // eblk),
        in_specs=[
            pl.BlockSpec((1, eblk, D), lambda bb, ii: (bb, ii, 0)),
            pl.BlockSpec((1, eblk, D), lambda bb, ii: (bb, ii, 0)),
            pl.BlockSpec((1, eblk, 16), lambda bb, ii: (bb, ii, 0)),
            pl.BlockSpec((1, D), lambda bb, ii: (0, 0)),
            pl.BlockSpec((1, D), lambda bb, ii: (0, 0)),
            pl.BlockSpec((1, 1), lambda bb, ii: (0, 0)),
            pl.BlockSpec((1, 1), lambda bb, ii: (0, 0)),
        ],
        out_specs=pl.BlockSpec((1, eblk, D), lambda bb, ii: (bb, ii, 0)),
        out_shape=jax.ShapeDtypeStruct((B, T, D), x.dtype),
    )(x, msgsum, idx, gain.reshape(1, D), bias.reshape(1, D),
      log_mix.reshape(1, 1), log_scale.reshape(1, 1))


# speedup vs baseline: 4.1338x; 4.1338x over previous
import functools

import jax
import jax.numpy as jnp
from jax.experimental import pallas as pl
from jax.experimental.pallas import tpu as pltpu

_K = 8
_NEG = -1e30
_NPATH = 8


def _fused_kernel(x_ref, gain_ref, bias_ref, lm_ref, ls_ref, out_ref, xn_ref,
                  *, blk):
    i = pl.program_id(1)

    @pl.when(i == 0)
    def _normalize():
        xa_full = x_ref[0]
        n2 = jnp.sum(xa_full * xa_full, axis=1, keepdims=True)
        xn_ref[...] = xa_full / (jnp.sqrt(n2) + 1e-8)

    row0 = i * blk
    mix = jax.nn.sigmoid(lm_ref[0, 0])
    scale = jax.nn.softplus(ls_ref[0, 0]) + 0.01

    def _path(wcols, static_tri):
        nmain = wcols - blk if static_tri else 0
        xn_rows = xn_ref[pl.ds(row0, blk), :]
        sim = jax.lax.dot_general(
            xn_rows, xn_ref[:wcols, :], (((1,), (1,)), ((), ())),
            preferred_element_type=jnp.float32)

        dcols = wcols - nmain
        cols = jax.lax.broadcasted_iota(jnp.int32, (blk, dcols), 1)
        rows = jax.lax.broadcasted_iota(jnp.int32, (blk, dcols), 0)
        if not static_tri:
            rows = row0 + rows
        wd = jnp.where(cols <= rows, sim[:, nmain:], _NEG)
        wm = sim[:, :nmain] if nmain else None

        def masked_max(arr, m):
            t = arr if m is None else jnp.where(arr < m, arr, _NEG)
            return jnp.max(t, axis=1, keepdims=True)

        m = None
        for _ in range(_K):
            md = masked_max(wd, m)
            m = jnp.maximum(masked_max(wm, m), md) if nmain else md
        thresh = jnp.maximum(m, -2.0)

        adj_d = jnp.where(wd >= thresh, 1.0, 0.0)
        deg = jnp.sum(adj_d, axis=1, keepdims=True)
        msg = jax.lax.dot_general(
            adj_d, x_ref[0, nmain:wcols, :], (((1,), (0,)), ((), ())),
            preferred_element_type=jnp.float32)
        if nmain:
            adj_m = jnp.where(wm >= thresh, 1.0, 0.0)
            deg = deg + jnp.sum(adj_m, axis=1, keepdims=True)
            msg = msg + jax.lax.dot_general(
                adj_m, x_ref[0, :nmain, :], (((1,), (0,)), ((), ())),
                preferred_element_type=jnp.float32)
        msg = msg / jnp.maximum(deg, 1.0)

        x_rows = x_ref[0, pl.ds(row0, blk), :]
        blended = mix * x_rows + (1.0 - mix) * msg
        y = blended * gain_ref[0][None, :] + bias_ref[0][None, :]
        gelu = 0.5 * y * (1.0 + jax.lax.erf(y * (2.0 ** -0.5)))
        out_ref[0] = gelu * scale

    t_total = x_ref.shape[1]
    nblk = t_total // blk
    if nblk <= _NPATH:
        for p in range(nblk):
            cond = (i == p) if p else (i < 1)

            @pl.when(cond)
            def _run(p=p):
                _path((p + 1) * blk, True)
    else:
        _path(t_total, False)


def kernel(x, gain, bias, log_mix, log_scale):
    B, T, D = x.shape
    blk = min(256, T)
    grid = (B, T // blk)

    fn = functools.partial(_fused_kernel, blk=blk)
    return pl.pallas_call(
        fn,
        grid=grid,
        in_specs=[
            pl.BlockSpec((1, T, D), lambda b, i: (b, 0, 0)),
            pl.BlockSpec((1, D), lambda b, i: (0, 0)),
            pl.BlockSpec((1, D), lambda b, i: (0, 0)),
            pl.BlockSpec((1, 1), lambda b, i: (0, 0)),
            pl.BlockSpec((1, 1), lambda b, i: (0, 0)),
        ],
        out_specs=pl.BlockSpec((1, blk, D), lambda b, i: (b, i, 0)),
        out_shape=jax.ShapeDtypeStruct((B, T, D), x.dtype),
        scratch_shapes=[pltpu.VMEM((T, D), jnp.float32)],
    )(x, gain.reshape(1, D), bias.reshape(1, D),
      log_mix.reshape(1, 1), log_scale.reshape(1, 1))
